# baseline (speedup 1.0000x reference)
import functools

import jax
import jax.numpy as jnp
from jax import lax
from jax.experimental import pallas as pl
from jax.experimental.pallas import tpu as pltpu
from jax.experimental.pallas import tpu_sc as plsc

_LAYER_W = 12500
_HID = 128

_NC, _NS = 2, 16
_NW = _NC * _NS
_CH, _CHW = 7, 128
_PAD_E = _NW * _CH * _CHW

_ROWS = 512


def _mm(x, w):
    return lax.dot_general(
        x.astype(jnp.bfloat16), w, (((1,), (1,)), ((), ())),
        preferred_element_type=jnp.float32)


def _resnet(x, w1, b1, w2, b2, w3, b3):
    h1 = jnp.tanh(_mm(x, w1) + b1)
    h2 = jnp.tanh(_mm(h1, w2) + b2)
    return _mm(h2 + x, w3) + b3


def _embed_body(x_ref, w_ref, b_ref, o_ref):
    o_ref[...] = jnp.tanh(_mm(x_ref[...], w_ref[...]) + b_ref[...])


def _embed(node_feats, w, b):
    n = node_feats.shape[0]
    return pl.pallas_call(
        _embed_body,
        grid=(pl.cdiv(n, _ROWS),),
        in_specs=[
            pl.BlockSpec((_ROWS, _HID), lambda i: (i, 0)),
            pl.BlockSpec((_HID, _HID), lambda i: (0, 0)),
            pl.BlockSpec((1, _HID), lambda i: (0, 0)),
        ],
        out_specs=pl.BlockSpec((_ROWS, _HID), lambda i: (i, 0)),
        out_shape=jax.ShapeDtypeStruct((n, _HID), jnp.float32),
    )(node_feats, w, b.reshape(1, _HID))


def _layer_body(msgs_ref, ope_ref, *refs):
    o_ref = refs[-1]
    w = [r[...] for r in refs[:-1]]
    ope = ope_ref[...]
    r = jnp.tanh(_resnet(msgs_ref[...], *w[0:6]))
    r = jnp.tanh(_resnet(r, *w[6:12]))
    w1b, b1, w2, b2, w3a, w3b, b3 = w[12:19]
    h1 = jnp.tanh(_mm(r, w1b) + b1)
    h2 = jnp.tanh(_mm(h1, w2) + b2)
    e = jnp.tanh(_mm(h2[:, :_HID], w3a) + _mm(h2[:, _HID:] + r, w3b) + b3)
    e = jnp.tanh(_resnet(e, *w[19:25]))
    w1a, w1b, b1, w2, b2, w3a, w3b, b3 = w[25:33]
    h1 = jnp.tanh(_mm(e, w1a) + _mm(ope, w1b) + b1)
    h2 = jnp.tanh(_mm(h1, w2) + b2)
    c = jnp.tanh(_mm(h2[:, :_HID] + e, w3a) + _mm(h2[:, _HID:] + ope, w3b) + b3)
    o_ref[...] = jnp.tanh(_resnet(c, *w[33:39]))


def _wb(p, lay):
    w = p[lay]["W"].astype(jnp.bfloat16)
    b = p[lay]["b"].reshape(1, -1)
    return w, b


def _layer_weights(packs):
    mp, mp1, ne, ne1, cb, cb1 = packs
    vals = []
    for p in (mp, mp1):
        for lay in ("lay1", "lay2", "lay3"):
            vals.extend(_wb(p, lay))
    w1, b1 = _wb(ne, "lay1")
    w2, b2 = _wb(ne, "lay2")
    w3, b3 = _wb(ne, "lay3")
    vals.extend([w1[:, _HID:], b1, w2, b2, w3[:, :_HID], w3[:, _HID:], b3])
    for lay in ("lay1", "lay2", "lay3"):
        vals.extend(_wb(ne1, lay))
    w1, b1 = _wb(cb, "lay1")
    w2, b2 = _wb(cb, "lay2")
    w3, b3 = _wb(cb, "lay3")
    vals.extend([w1[:, :_HID], w1[:, _HID:], b1, w2, b2,
                 w3[:, :_HID], w3[:, _HID:], b3])
    for lay in ("lay1", "lay2", "lay3"):
        vals.extend(_wb(cb1, lay))
    return vals


def _layer(msgs, ope, packs):
    wvals = _layer_weights(packs)
    wspecs = [pl.BlockSpec(v.shape, lambda i, n=v.ndim: (0,) * n)
              for v in wvals]
    return pl.pallas_call(
        _layer_body,
        grid=(pl.cdiv(_LAYER_W, _ROWS),),
        in_specs=[
            pl.BlockSpec((_ROWS, 2 * _HID), lambda i: (i, 0)),
            pl.BlockSpec((_ROWS, _HID), lambda i: (i, 0)),
        ] + wspecs,
        out_specs=pl.BlockSpec((_ROWS, _HID), lambda i: (i, 0)),
        out_shape=jax.ShapeDtypeStruct((_LAYER_W, _HID), jnp.float32),
    )(msgs, ope, *wvals)


def _gather_body(table_hbm, idx_hbm, out_hbm, idx_v, rows_v, sem):
    wid = lax.axis_index("s") * _NC + lax.axis_index("c")
    pltpu.sync_copy(idx_hbm.at[wid], idx_v)
    cps = [pltpu.async_copy(table_hbm.at[idx_v.at[j]], rows_v.at[j], sem)
           for j in range(_CH)]
    for c in cps:
        c.wait()
    pltpu.sync_copy(rows_v, out_hbm.at[wid])


@functools.cache
def _gather_k():
    mesh = plsc.VectorSubcoreMesh(
        core_axis_name="c", subcore_axis_name="s",
        num_cores=_NC, num_subcores=_NS)
    return pl.kernel(
        _gather_body,
        out_type=jax.ShapeDtypeStruct((_NW, _CH, _CHW, _HID), jnp.float32),
        mesh=mesh,
        scratch_types=[
            pltpu.VMEM((_CH, _CHW), jnp.int32),
            pltpu.VMEM((_CH, _CHW, _HID), jnp.float32),
            pltpu.SemaphoreType.DMA,
        ],
    )


def kernel(node_feats, params, src_idx):
    emb = params["embed_op"]
    ope = _embed(node_feats, emb["W"].astype(jnp.bfloat16), emb["b"])

    picks = [
        ("d1_mp_binary", "d1_node_embeds", "d1_comb_embed"),
        ("d2_mp_binary", "d2_node_embeds", "d2_comb_embed"),
        ("d2_mp_binary", "d3_node_embeds", "d2_comb_embed"),
    ]
    outs = [ope[:_LAYER_W]]
    prev = outs[0]
    for layer in range(1, 4):
        mp, ne, cb = picks[layer - 1]
        loc = src_idx[(layer - 1) * _LAYER_W: layer * _LAYER_W]
        flat = loc.reshape(-1)
        flat = jnp.concatenate(
            [flat, jnp.zeros((_PAD_E - flat.shape[0],), jnp.int32)])
        idx3 = flat.reshape(_NW, _CH, _CHW)
        rows = _gather_k()(prev, idx3)
        msgs = rows.reshape(_PAD_E // 2, 2 * _HID)
        opeb = lax.slice_in_dim(ope, layer * _LAYER_W, (layer + 1) * _LAYER_W)
        packs = [params[mp], params[mp + "1"],
                 params[ne], params[ne + "1"],
                 params[cb], params[cb + "1"]]
        prev = _layer(msgs, opeb, packs)
        outs.append(prev)
    return jnp.concatenate(outs, axis=0)


# speedup vs baseline: 1.1967x; 1.1967x over previous
import functools

import jax
import jax.numpy as jnp
from jax import lax
from jax.experimental import pallas as pl
from jax.experimental.pallas import tpu as pltpu
from jax.experimental.pallas import tpu_sc as plsc

_LAYER_W = 12500
_HID = 128

_NC, _NS = 2, 16
_NW = _NC * _NS
_CH, _CHW = 7, 128
_PAD_E = _NW * _CH * _CHW
_TBL_PAD = 12544

_ROWS = 1792
_BF = jnp.bfloat16


def _mm(x, w):
    return lax.dot_general(
        x, w, (((1,), (1,)), ((), ())),
        preferred_element_type=jnp.float32).astype(_BF)


def _resnet(x, w1, b1, w2, b2, w3, b3):
    h1 = jnp.tanh(_mm(x, w1) + b1)
    h2 = jnp.tanh(_mm(h1, w2) + b2)
    return _mm(h2 + x, w3) + b3


def _embed_body(x_ref, w_ref, b_ref, o_ref):
    o_ref[...] = jnp.tanh(_mm(x_ref[...].astype(_BF), w_ref[...]) + b_ref[...])


def _embed(node_feats, w, b):
    n = node_feats.shape[0]
    return pl.pallas_call(
        _embed_body,
        grid=(pl.cdiv(n, _ROWS),),
        in_specs=[
            pl.BlockSpec((_ROWS, _HID), lambda i: (i, 0)),
            pl.BlockSpec((_HID, _HID), lambda i: (0, 0)),
            pl.BlockSpec((1, _HID), lambda i: (0, 0)),
        ],
        out_specs=pl.BlockSpec((_ROWS, _HID), lambda i: (i, 0)),
        out_shape=jax.ShapeDtypeStruct((n, _HID), _BF),
    )(node_feats, w, b.reshape(1, _HID))


def _layer_compute(msgs, ope, w):
    r = jnp.tanh(_resnet(msgs, *w[0:6]))
    r = jnp.tanh(_resnet(r, *w[6:12]))
    w1b, b1, w2, b2, w3a, w3b, b3 = w[12:19]
    h1 = jnp.tanh(_mm(r, w1b) + b1)
    h2 = jnp.tanh(_mm(h1, w2) + b2)
    e = jnp.tanh(_mm(h2[:, :_HID], w3a) + _mm(h2[:, _HID:] + r, w3b) + b3)
    e = jnp.tanh(_resnet(e, *w[19:25]))
    w1a, w1b, b1, w2, b2, w3a, w3b, b3 = w[25:33]
    h1 = jnp.tanh(_mm(e, w1a) + _mm(ope, w1b) + b1)
    h2 = jnp.tanh(_mm(h1, w2) + b2)
    c = jnp.tanh(_mm(h2[:, :_HID] + e, w3a) + _mm(h2[:, _HID:] + ope, w3b) + b3)
    return jnp.tanh(_resnet(c, *w[33:39]))


def _layer_body(msgs_ref, ope_ref, *refs):
    o32_ref = refs[-1]
    w = [r[...] for r in refs[:-1]]
    out = _layer_compute(msgs_ref[...].astype(_BF), ope_ref[...], w)
    o32_ref[...] = out.astype(jnp.float32)


def _wb(p, lay):
    w = p[lay]["W"].astype(_BF)
    b = p[lay]["b"].astype(_BF).reshape(1, -1)
    return w, b


def _layer_weights(packs):
    mp, mp1, ne, ne1, cb, cb1 = packs
    vals = []
    for p in (mp, mp1):
        for lay in ("lay1", "lay2", "lay3"):
            vals.extend(_wb(p, lay))
    w1, b1 = _wb(ne, "lay1")
    w2, b2 = _wb(ne, "lay2")
    w3, b3 = _wb(ne, "lay3")
    vals.extend([w1[:, _HID:], b1, w2, b2, w3[:, :_HID], w3[:, _HID:], b3])
    for lay in ("lay1", "lay2", "lay3"):
        vals.extend(_wb(ne1, lay))
    w1, b1 = _wb(cb, "lay1")
    w2, b2 = _wb(cb, "lay2")
    w3, b3 = _wb(cb, "lay3")
    vals.extend([w1[:, :_HID], w1[:, _HID:], b1, w2, b2,
                 w3[:, :_HID], w3[:, _HID:], b3])
    for lay in ("lay1", "lay2", "lay3"):
        vals.extend(_wb(cb1, lay))
    return vals


def _layer(msgs, ope, packs):
    wvals = _layer_weights(packs)
    wspecs = [pl.BlockSpec(v.shape, lambda i, n=v.ndim: (0,) * n)
              for v in wvals]
    return pl.pallas_call(
        _layer_body,
        grid=(pl.cdiv(_LAYER_W, _ROWS),),
        in_specs=[
            pl.BlockSpec((_ROWS, 2 * _HID), lambda i: (i, 0)),
            pl.BlockSpec((_ROWS, _HID), lambda i: (i, 0)),
        ] + wspecs,
        out_specs=pl.BlockSpec((_ROWS, _HID), lambda i: (i, 0)),
        out_shape=jax.ShapeDtypeStruct((_TBL_PAD, _HID), jnp.float32),
    )(msgs, ope, *wvals)


def _gather_body(table_hbm, idx_hbm, out_hbm, idx_v, rows_v, sem):
    cid = lax.axis_index("c")
    sid = lax.axis_index("s")
    pltpu.sync_copy(idx_hbm.at[cid, sid], idx_v)
    cps = [pltpu.async_copy(table_hbm.at[idx_v.at[j]],
                            rows_v.at[pl.ds(j * _CHW, _CHW)], sem)
           for j in range(_CH)]
    for c in cps:
        c.wait()
    pltpu.sync_copy(rows_v, out_hbm.at[cid, sid])


@functools.cache
def _gather_k():
    mesh = plsc.VectorSubcoreMesh(
        core_axis_name="c", subcore_axis_name="s",
        num_cores=_NC, num_subcores=_NS)
    return pl.kernel(
        _gather_body,
        out_type=jax.ShapeDtypeStruct(
            (_NC, _NS, _CH * _CHW, _HID), jnp.float32),
        mesh=mesh,
        scratch_types=[
            pltpu.VMEM((_CH, _CHW), jnp.int32),
            pltpu.VMEM((_CH * _CHW, _HID), jnp.float32),
            pltpu.SemaphoreType.DMA,
        ],
    )


def kernel(node_feats, params, src_idx):
    emb = params["embed_op"]
    ope = _embed(node_feats, emb["W"].astype(_BF),
                 emb["b"].astype(_BF))

    picks = [
        ("d1_mp_binary", "d1_node_embeds", "d1_comb_embed"),
        ("d2_mp_binary", "d2_node_embeds", "d2_comb_embed"),
        ("d2_mp_binary", "d3_node_embeds", "d2_comb_embed"),
    ]
    prev = jnp.pad(ope[:_LAYER_W].astype(jnp.float32),
                   ((0, _TBL_PAD - _LAYER_W), (0, 0)))
    outs = [prev[:_LAYER_W]]
    for layer in range(1, 4):
        mp, ne, cb = picks[layer - 1]
        loc = src_idx[(layer - 1) * _LAYER_W: layer * _LAYER_W]
        flat = loc.reshape(-1)
        flat = jnp.concatenate(
            [flat, jnp.zeros((_PAD_E - flat.shape[0],), jnp.int32)])
        idx4 = flat.reshape(_NC, _NS, _CH, _CHW)
        rows = _gather_k()(prev, idx4)
        msgs = rows.reshape(_PAD_E // 2, 2 * _HID)
        opeb = lax.slice_in_dim(ope, layer * _LAYER_W, (layer + 1) * _LAYER_W)
        packs = [params[mp], params[mp + "1"],
                 params[ne], params[ne + "1"],
                 params[cb], params[cb + "1"]]
        prev = _layer(msgs, opeb, packs)
        outs.append(prev[:_LAYER_W])
    return jnp.concatenate(outs, axis=0)
